# baseline (device time: 282681 ns/iter reference)
import jax
import jax.numpy as jnp
from jax import lax
from jax.experimental import pallas as pl
from jax.experimental.pallas import tpu as pltpu

N_DEV = 16
SQ = 1024
SKV_LOC = 1024
HQ = 8
DH = 128
DM = HQ * DH
BLK = 64
W = DM + 128
CHUNK = SQ // N_DEV
SCALE = 0.08838834764831843
KV_BLOCKS_LOC = SKV_LOC // BLK


def _attn_allreduce(q, k, v):

    def body(q_ref, k_ref, v_ref, acc_ref, comm_ref, send_sem, recv_sem,
             credit_sem):
        p = lax.axis_index("i")
        left = lax.rem(p + N_DEV - 1, N_DEV)
        right = lax.rem(p + 1, N_DEV)

        barrier = pltpu.get_barrier_semaphore()
        for nbr in (left, right):
            pl.semaphore_signal(barrier, inc=1, device_id=(nbr,),
                                device_id_type=pl.DeviceIdType.MESH)
        pl.semaphore_wait(barrier, 2)

        rows = lax.broadcasted_iota(jnp.int32, (SQ, SKV_LOC), 0)
        cols = lax.broadcasted_iota(jnp.int32, (SQ, SKV_LOC), 1)
        qb = rows // BLK
        jg = p * KV_BLOCKS_LOC + cols // BLK
        mask = (qb == jg) | (jg == 0) | (lax.rem(qb + jg, 3) == 0)

        ls = []
        for h in range(HQ):
            qh = q_ref[:, h * DH:(h + 1) * DH]
            kh = k_ref[:, h * DH:(h + 1) * DH]
            vh = v_ref[:, h * DH:(h + 1) * DH]
            s = lax.dot_general(qh, kh, (((1,), (1,)), ((), ())),
                                preferred_element_type=jnp.float32)
            w = jnp.where(mask, jnp.exp(s * SCALE), 0.0)
            ls.append(jnp.sum(w, axis=1, keepdims=True))
            acc_ref[:, h * DH:(h + 1) * DH] = jnp.dot(
                w.astype(jnp.bfloat16), vh,
                preferred_element_type=jnp.float32)
        lmat = jnp.concatenate(
            ls + [jnp.zeros((SQ, 128 - HQ), jnp.float32)], axis=1)
        acc_ref[:, DM:] = lmat

        n_steps = 2 * (N_DEV - 1)
        for g in range(n_steps):
            if g < N_DEV - 1:
                send_c = lax.rem(p - g + 2 * N_DEV, N_DEV)
                recv_c = lax.rem(p - g - 1 + 2 * N_DEV, N_DEV)
            else:
                t = g - (N_DEV - 1)
                send_c = lax.rem(p + 1 - t + 2 * N_DEV, N_DEV)
                recv_c = lax.rem(p - t + 2 * N_DEV, N_DEV)
            if g > 0:
                pl.semaphore_wait(credit_sem, 1)
            rdma = pltpu.make_async_remote_copy(
                src_ref=acc_ref.at[pl.ds(send_c * CHUNK, CHUNK), :],
                dst_ref=comm_ref,
                send_sem=send_sem,
                recv_sem=recv_sem,
                device_id=(right,),
                device_id_type=pl.DeviceIdType.MESH,
            )
            rdma.start()
            rdma.wait_send()
            rdma.wait_recv()
            dst = pl.ds(recv_c * CHUNK, CHUNK)
            if g < N_DEV - 1:
                acc_ref[dst, :] = acc_ref[dst, :] + comm_ref[:, :]
            else:
                acc_ref[dst, :] = comm_ref[:, :]
            if g < n_steps - 1:
                pl.semaphore_signal(credit_sem, inc=1, device_id=(left,),
                                    device_id_type=pl.DeviceIdType.MESH)

    return pl.pallas_call(
        body,
        out_shape=jax.ShapeDtypeStruct((SQ, W), jnp.float32),
        in_specs=[pl.BlockSpec(memory_space=pltpu.VMEM)] * 3,
        out_specs=pl.BlockSpec(memory_space=pltpu.VMEM),
        scratch_shapes=[
            pltpu.VMEM((CHUNK, W), jnp.float32),
            pltpu.SemaphoreType.DMA,
            pltpu.SemaphoreType.DMA,
            pltpu.SemaphoreType.REGULAR,
        ],
        compiler_params=pltpu.CompilerParams(collective_id=0),
    )(q, k, v)


def kernel(x, Wq, K_ext, V_ext, Wo):
    q = jnp.dot(x[0].astype(jnp.bfloat16), Wq.astype(jnp.bfloat16),
                preferred_element_type=jnp.bfloat16)
    k = K_ext[0].reshape(SKV_LOC, DM).astype(jnp.bfloat16)
    v = V_ext[0].reshape(SKV_LOC, DM).astype(jnp.bfloat16)

    acc = _attn_allreduce(q, k, v)
    ctx = acc[:, :DM]
    l = acc[:, DM:DM + HQ]
    norm = ctx / jnp.repeat(l, DH, axis=1)
    out = jnp.dot(norm.astype(jnp.bfloat16), Wo.astype(jnp.bfloat16),
                  preferred_element_type=jnp.float32)
    return out.reshape(1, SQ, DM)


# device time: 149435 ns/iter; 1.8917x vs baseline; 1.8917x over previous
import jax
import jax.numpy as jnp
from jax import lax
from jax.experimental import pallas as pl
from jax.experimental.pallas import tpu as pltpu

N_DEV = 16
SQ = 1024
SKV_LOC = 1024
HQ = 8
DH = 128
DM = HQ * DH
BLK = 64
W = DM + 128
SCALE = 0.08838834764831843
KV_BLOCKS_LOC = SKV_LOC // BLK

MASKS_RS = [1, 4, 2, 8]
SHIFTS_RS = [0, 2, 1, 3]


def _attn_allreduce(q, k, v):

    def body(q_ref, k_ref, v_ref, acc_ref,
             land0, land1, land2, land3, send_sem, recv_sems):
        p = lax.axis_index("i")

        barrier = pltpu.get_barrier_semaphore()
        for m in MASKS_RS:
            pl.semaphore_signal(barrier, inc=1, device_id=(p ^ m,),
                                device_id_type=pl.DeviceIdType.MESH)
        pl.semaphore_wait(barrier, 4)

        rows = lax.broadcasted_iota(jnp.int32, (SQ, SKV_LOC), 0)
        cols = lax.broadcasted_iota(jnp.int32, (SQ, SKV_LOC), 1)
        qb = rows // BLK
        jg = p * KV_BLOCKS_LOC + cols // BLK
        mask = (qb == jg) | (jg == 0) | (lax.rem(qb + jg, 3) == 0)

        ls = []
        for h in range(HQ):
            qh = q_ref[:, h * DH:(h + 1) * DH]
            kh = k_ref[:, h * DH:(h + 1) * DH]
            vh = v_ref[:, h * DH:(h + 1) * DH]
            s = lax.dot_general(qh, kh, (((1,), (1,)), ((), ())),
                                preferred_element_type=jnp.float32)
            w = jnp.where(mask, jnp.exp(s * SCALE), 0.0)
            ls.append(jnp.sum(w, axis=1, keepdims=True))
            acc_ref[:, h * DH:(h + 1) * DH] = jnp.dot(
                w.astype(jnp.bfloat16), vh,
                preferred_element_type=jnp.float32)
        lmat = jnp.concatenate(
            ls + [jnp.zeros((SQ, 128 - HQ), jnp.float32)], axis=1)
        acc_ref[:, DM:] = lmat

        lands = [land0, land1, land2, land3]

        start = jnp.int32(0)
        size = SQ
        for j, (m, sh) in enumerate(zip(MASKS_RS, SHIFTS_RS)):
            half = size // 2
            b = (p >> sh) & 1
            keep_start = pl.multiple_of(start + b * half, 64)
            send_start = pl.multiple_of(start + (1 - b) * half, 64)
            rdma = pltpu.make_async_remote_copy(
                src_ref=acc_ref.at[pl.ds(send_start, half)],
                dst_ref=lands[j],
                send_sem=send_sem,
                recv_sem=recv_sems.at[j],
                device_id=(p ^ m,),
                device_id_type=pl.DeviceIdType.MESH,
            )
            rdma.start()
            rdma.wait_send()
            rdma.wait_recv()
            acc_ref[pl.ds(keep_start, half)] = (
                acc_ref[pl.ds(keep_start, half)] + lands[j][...])
            start = keep_start
            size = half

        for j, m in enumerate(reversed(MASKS_RS)):
            parent = pl.multiple_of(start - lax.rem(start, 2 * size), 64)
            partner_start = pl.multiple_of(
                parent + (size - (start - parent)), 64)
            start = pl.multiple_of(start, 64)
            send = pltpu.make_async_remote_copy(
                src_ref=acc_ref.at[pl.ds(start, size)],
                dst_ref=acc_ref.at[pl.ds(start, size)],
                send_sem=send_sem,
                recv_sem=recv_sems.at[4 + j],
                device_id=(p ^ m,),
                device_id_type=pl.DeviceIdType.MESH,
            )
            send.start()
            send.wait_send()
            recv = pltpu.make_async_remote_copy(
                src_ref=acc_ref.at[pl.ds(start, size)],
                dst_ref=acc_ref.at[pl.ds(partner_start, size)],
                send_sem=send_sem,
                recv_sem=recv_sems.at[4 + j],
                device_id=(p ^ m,),
                device_id_type=pl.DeviceIdType.MESH,
            )
            recv.wait_recv()
            start = parent
            size = 2 * size

    return pl.pallas_call(
        body,
        out_shape=jax.ShapeDtypeStruct((SQ, W), jnp.float32),
        in_specs=[pl.BlockSpec(memory_space=pltpu.VMEM)] * 3,
        out_specs=pl.BlockSpec(memory_space=pltpu.VMEM),
        scratch_shapes=[
            pltpu.VMEM((SQ // 2, W), jnp.float32),
            pltpu.VMEM((SQ // 4, W), jnp.float32),
            pltpu.VMEM((SQ // 8, W), jnp.float32),
            pltpu.VMEM((SQ // 16, W), jnp.float32),
            pltpu.SemaphoreType.DMA,
            pltpu.SemaphoreType.DMA((8,)),
        ],
        compiler_params=pltpu.CompilerParams(collective_id=0),
    )(q, k, v)


def kernel(x, Wq, K_ext, V_ext, Wo):
    q = jnp.dot(x[0].astype(jnp.bfloat16), Wq.astype(jnp.bfloat16),
                preferred_element_type=jnp.bfloat16)
    k = K_ext[0].reshape(SKV_LOC, DM).astype(jnp.bfloat16)
    v = V_ext[0].reshape(SKV_LOC, DM).astype(jnp.bfloat16)

    acc = _attn_allreduce(q, k, v)
    ctx = acc[:, :DM]
    l = acc[:, DM:DM + HQ]
    norm = ctx / jnp.repeat(l, DH, axis=1)
    out = jnp.dot(norm.astype(jnp.bfloat16), Wo.astype(jnp.bfloat16),
                  preferred_element_type=jnp.float32)
    return out.reshape(1, SQ, DM)


# device time: 92641 ns/iter; 3.0514x vs baseline; 1.6131x over previous
import jax
import jax.numpy as jnp
from jax import lax
from jax.experimental import pallas as pl
from jax.experimental.pallas import tpu as pltpu

N_DEV = 16
SQ = 1024
SKV_LOC = 1024
HQ = 8
DH = 128
DM = HQ * DH
BLK = 64
W = DM + 128
SCALE = 0.08838834764831843
KV_BLOCKS_LOC = SKV_LOC // BLK

MASKS_RS = [1, 4, 2]
SHIFTS_RS = [0, 2, 1]
MASKS_AG = [2, 4, 1]


def _attn_allreduce(q, k, v):

    def body(q_ref, k_ref, v_ref, abuf_ref, acc_ref,
             land0, land1, land2, land3, send_sem, recv_sems):
        p = lax.axis_index("i")

        barrier = pltpu.get_barrier_semaphore()
        for m in (1, 2, 4, 8):
            pl.semaphore_signal(barrier, inc=1, device_id=(p ^ m,),
                                device_id_type=pl.DeviceIdType.MESH)
        pl.semaphore_wait(barrier, 4)

        def compute_band(off, n):
            rows = lax.broadcasted_iota(jnp.int32, (n, SKV_LOC), 0) + off
            cols = lax.broadcasted_iota(jnp.int32, (n, SKV_LOC), 1)
            qb = rows // BLK
            jg = p * KV_BLOCKS_LOC + cols // BLK
            mask = (qb == jg) | (jg == 0) | (lax.rem(qb + jg, 3) == 0)
            ls = []
            for h in range(HQ):
                qh = q_ref[pl.ds(off, n), pl.ds(h * DH, DH)]
                kh = k_ref[:, h * DH:(h + 1) * DH]
                vh = v_ref[:, h * DH:(h + 1) * DH]
                s = lax.dot_general(qh, kh, (((1,), (1,)), ((), ())),
                                    preferred_element_type=jnp.float32)
                w = jnp.where(mask, jnp.exp(s * SCALE), 0.0)
                ls.append(jnp.sum(w, axis=1, keepdims=True))
                acc_ref[pl.ds(off, n), pl.ds(h * DH, DH)] = jnp.dot(
                    w.astype(jnp.bfloat16), vh,
                    preferred_element_type=jnp.float32)
            acc_ref[pl.ds(off, n), pl.ds(DM, 128)] = jnp.concatenate(
                ls + [jnp.zeros((n, 128 - HQ), jnp.float32)], axis=1)

        lands = [land0, land1, land2]
        prev_send = None

        b0 = p & 1
        send_start = pl.multiple_of((1 - b0) * 512, 64)
        keep_start = pl.multiple_of(b0 * 512, 64)
        compute_band(send_start, 512)
        abuf_ref[pl.ds(send_start, 512)] = (
            acc_ref[pl.ds(send_start, 512)].astype(jnp.bfloat16))
        rdma = pltpu.make_async_remote_copy(
            src_ref=abuf_ref.at[pl.ds(send_start, 512)],
            dst_ref=land0,
            send_sem=send_sem,
            recv_sem=recv_sems.at[0],
            device_id=(p ^ 1,),
            device_id_type=pl.DeviceIdType.MESH,
        )
        rdma.start()
        prev_send = rdma
        compute_band(keep_start, 512)
        rdma.wait_recv()
        acc_ref[pl.ds(keep_start, 512)] = (
            acc_ref[pl.ds(keep_start, 512)] + land0[...].astype(jnp.float32))
        start = keep_start
        size = 512

        for j, (m, sh) in enumerate(zip(MASKS_RS[1:], SHIFTS_RS[1:]), 1):
            half = size // 2
            b = (p >> sh) & 1
            keep_start = pl.multiple_of(start + b * half, 64)
            send_start = pl.multiple_of(start + (1 - b) * half, 64)
            abuf_ref[pl.ds(send_start, half)] = (
                acc_ref[pl.ds(send_start, half)].astype(jnp.bfloat16))
            rdma = pltpu.make_async_remote_copy(
                src_ref=abuf_ref.at[pl.ds(send_start, half)],
                dst_ref=lands[j],
                send_sem=send_sem,
                recv_sem=recv_sems.at[j],
                device_id=(p ^ m,),
                device_id_type=pl.DeviceIdType.MESH,
            )
            prev_send.wait_send()
            rdma.start()
            prev_send = rdma
            rdma.wait_recv()
            acc_ref[pl.ds(keep_start, half)] = (
                acc_ref[pl.ds(keep_start, half)]
                + lands[j][...].astype(jnp.float32))
            start = keep_start
            size = half

        start = pl.multiple_of(start, 64)
        abuf_ref[pl.ds(start, 128)] = (
            acc_ref[pl.ds(start, 128)].astype(jnp.bfloat16))
        rdma = pltpu.make_async_remote_copy(
            src_ref=abuf_ref.at[pl.ds(start, 128)],
            dst_ref=land3,
            send_sem=send_sem,
            recv_sem=recv_sems.at[3],
            device_id=(p ^ 8,),
            device_id_type=pl.DeviceIdType.MESH,
        )
        prev_send.wait_send()
        rdma.start()
        prev_send = rdma
        rdma.wait_recv()
        abuf_ref[pl.ds(start, 128)] = (
            acc_ref[pl.ds(start, 128)] + land3[...].astype(jnp.float32)
        ).astype(jnp.bfloat16)
        size = 128

        for j, m in enumerate(MASKS_AG):
            parent = pl.multiple_of(start - lax.rem(start, 2 * size), 64)
            partner_start = pl.multiple_of(
                parent + (size - (start - parent)), 64)
            send = pltpu.make_async_remote_copy(
                src_ref=abuf_ref.at[pl.ds(start, size)],
                dst_ref=abuf_ref.at[pl.ds(start, size)],
                send_sem=send_sem,
                recv_sem=recv_sems.at[4 + j],
                device_id=(p ^ m,),
                device_id_type=pl.DeviceIdType.MESH,
            )
            prev_send.wait_send()
            send.start()
            prev_send = send
            recv = pltpu.make_async_remote_copy(
                src_ref=abuf_ref.at[pl.ds(start, size)],
                dst_ref=abuf_ref.at[pl.ds(partner_start, size)],
                send_sem=send_sem,
                recv_sem=recv_sems.at[4 + j],
                device_id=(p ^ m,),
                device_id_type=pl.DeviceIdType.MESH,
            )
            recv.wait_recv()
            start = parent
            size = 2 * size

        prev_send.wait_send()

    return pl.pallas_call(
        body,
        out_shape=jax.ShapeDtypeStruct((SQ, W), jnp.bfloat16),
        in_specs=[pl.BlockSpec(memory_space=pltpu.VMEM)] * 3,
        out_specs=pl.BlockSpec(memory_space=pltpu.VMEM),
        scratch_shapes=[
            pltpu.VMEM((SQ, W), jnp.float32),
            pltpu.VMEM((512, W), jnp.bfloat16),
            pltpu.VMEM((256, W), jnp.bfloat16),
            pltpu.VMEM((128, W), jnp.bfloat16),
            pltpu.VMEM((128, W), jnp.bfloat16),
            pltpu.SemaphoreType.DMA,
            pltpu.SemaphoreType.DMA((7,)),
        ],
        compiler_params=pltpu.CompilerParams(collective_id=0),
    )(q, k, v)


def kernel(x, Wq, K_ext, V_ext, Wo):
    q = jnp.dot(x[0].astype(jnp.bfloat16), Wq.astype(jnp.bfloat16),
                preferred_element_type=jnp.bfloat16)
    k = K_ext[0].reshape(SKV_LOC, DM).astype(jnp.bfloat16)
    v = V_ext[0].reshape(SKV_LOC, DM).astype(jnp.bfloat16)

    a = _attn_allreduce(q, k, v)
    ctx = a[:, :DM].astype(jnp.float32)
    l = a[:, DM:DM + HQ].astype(jnp.float32)
    norm = ctx / jnp.repeat(l, DH, axis=1)
    out = jnp.dot(norm.astype(jnp.bfloat16), Wo.astype(jnp.bfloat16),
                  preferred_element_type=jnp.float32)
    return out.reshape(1, SQ, DM)


# device time: 83422 ns/iter; 3.3886x vs baseline; 1.1105x over previous
import jax
import jax.numpy as jnp
from jax import lax
from jax.experimental import pallas as pl
from jax.experimental.pallas import tpu as pltpu

N_DEV = 16
SQ = 1024
SKV_LOC = 1024
HQ = 8
DH = 128
DM = HQ * DH
BLK = 64
W = DM + 128
SCALE = 0.08838834764831843
KV_BLOCKS_LOC = SKV_LOC // BLK

MASKS_RS = [1, 4, 2]
SHIFTS_RS = [0, 2, 1]
MASKS_AG = [2, 4, 1]
BF16 = jnp.bfloat16
F32 = jnp.float32


def _fused(x2d, wq, k, v, wo):

    def body(x_ref, wq_ref, k_ref, v_ref, wo_ref, out_ref,
             qbuf, acc_ref, abuf_ref, land0, land1, land2, land3,
             rs_sem, ag_sems, recv_sems):
        p = lax.axis_index("i")

        barrier = pltpu.get_barrier_semaphore()
        for m in (1, 2, 4, 8):
            pl.semaphore_signal(barrier, inc=1, device_id=(p ^ m,),
                                device_id_type=pl.DeviceIdType.MESH)

        qbuf[...] = jnp.dot(x_ref[...], wq_ref[...],
                            preferred_element_type=F32).astype(BF16)

        def compute_band(off, n):
            rows = lax.broadcasted_iota(jnp.int32, (n, SKV_LOC), 0) + off
            cols = lax.broadcasted_iota(jnp.int32, (n, SKV_LOC), 1)
            qb = rows // BLK
            jg = p * KV_BLOCKS_LOC + cols // BLK
            mask = (qb == jg) | (jg == 0) | (lax.rem(qb + jg, 3) == 0)
            ls = []
            for h in range(HQ):
                qh = qbuf[pl.ds(off, n), pl.ds(h * DH, DH)]
                kh = k_ref[:, h * DH:(h + 1) * DH]
                vh = v_ref[:, h * DH:(h + 1) * DH]
                s = lax.dot_general(qh, kh, (((1,), (1,)), ((), ())),
                                    preferred_element_type=F32)
                w = jnp.where(mask, jnp.exp(s * SCALE), 0.0)
                ls.append(jnp.sum(w, axis=1, keepdims=True))
                acc_ref[pl.ds(off, n), pl.ds(h * DH, DH)] = jnp.dot(
                    w.astype(BF16), vh, preferred_element_type=F32)
            acc_ref[pl.ds(off, n), pl.ds(DM, 128)] = jnp.concatenate(
                ls + [jnp.zeros((n, 128 - HQ), F32)], axis=1)

        def out_band(off, n):
            lb = abuf_ref[pl.ds(off, n), pl.ds(DM, 128)].astype(F32)
            o = jnp.zeros((n, DM), F32)
            for h in range(HQ):
                ch = abuf_ref[pl.ds(off, n), pl.ds(h * DH, DH)].astype(F32)
                nh = (ch / lb[:, h:h + 1]).astype(BF16)
                o = o + jnp.dot(nh, wo_ref[h * DH:(h + 1) * DH, :],
                                preferred_element_type=F32)
            out_ref[pl.ds(off, n), :] = o

        b0 = p & 1
        send_start = pl.multiple_of((1 - b0) * 512, 64)
        keep_start = pl.multiple_of(b0 * 512, 64)
        compute_band(send_start, 512)
        abuf_ref[pl.ds(send_start, 512)] = (
            acc_ref[pl.ds(send_start, 512)].astype(BF16))
        pl.semaphore_wait(barrier, 4)
        rs0 = pltpu.make_async_remote_copy(
            src_ref=abuf_ref.at[pl.ds(send_start, 512)],
            dst_ref=land0,
            send_sem=rs_sem,
            recv_sem=recv_sems.at[0],
            device_id=(p ^ 1,),
            device_id_type=pl.DeviceIdType.MESH,
        )
        rs0.start()
        compute_band(keep_start, 512)
        rs0.wait_recv()
        acc_ref[pl.ds(keep_start, 512)] = (
            acc_ref[pl.ds(keep_start, 512)] + land0[...].astype(F32))
        prev_send = rs0
        start = keep_start
        size = 512

        lands = [land0, land1, land2]
        for j, (m, sh) in enumerate(zip(MASKS_RS[1:], SHIFTS_RS[1:]), 1):
            half = size // 2
            b = (p >> sh) & 1
            keep_start = pl.multiple_of(start + b * half, 64)
            send_start = pl.multiple_of(start + (1 - b) * half, 64)
            abuf_ref[pl.ds(send_start, half)] = (
                acc_ref[pl.ds(send_start, half)].astype(BF16))
            rdma = pltpu.make_async_remote_copy(
                src_ref=abuf_ref.at[pl.ds(send_start, half)],
                dst_ref=lands[j],
                send_sem=rs_sem,
                recv_sem=recv_sems.at[j],
                device_id=(p ^ m,),
                device_id_type=pl.DeviceIdType.MESH,
            )
            prev_send.wait_send()
            rdma.start()
            prev_send = rdma
            rdma.wait_recv()
            acc_ref[pl.ds(keep_start, half)] = (
                acc_ref[pl.ds(keep_start, half)]
                + lands[j][...].astype(F32))
            start = keep_start
            size = half

        start = pl.multiple_of(start, 64)
        abuf_ref[pl.ds(start, 128)] = acc_ref[pl.ds(start, 128)].astype(BF16)
        prev_send.wait_send()
        xadd = pltpu.make_async_remote_copy(
            src_ref=abuf_ref.at[pl.ds(start, 128)],
            dst_ref=land3,
            send_sem=rs_sem,
            recv_sem=recv_sems.at[3],
            device_id=(p ^ 8,),
            device_id_type=pl.DeviceIdType.MESH,
        )
        xadd.start()
        xadd.wait_recv()
        xadd.wait_send()
        abuf_ref[pl.ds(start, 128)] = (
            acc_ref[pl.ds(start, 128)] + land3[...].astype(F32)).astype(BF16)

        r0 = start
        parent1 = pl.multiple_of(r0 - lax.rem(r0, 256), 64)
        p0 = pl.multiple_of(parent1 + (128 - (r0 - parent1)), 64)
        parent2 = pl.multiple_of(parent1 - lax.rem(parent1, 512), 64)
        p1 = pl.multiple_of(parent2 + (256 - (parent1 - parent2)), 64)
        parent3 = pl.multiple_of(parent2 - lax.rem(parent2, 1024), 64)
        p2 = pl.multiple_of(parent3 + (512 - (parent2 - parent3)), 64)

        def ag_send(src_off, n, partner_m, sem_idx, recv_idx):
            d = pltpu.make_async_remote_copy(
                src_ref=abuf_ref.at[pl.ds(src_off, n)],
                dst_ref=abuf_ref.at[pl.ds(src_off, n)],
                send_sem=ag_sems.at[sem_idx],
                recv_sem=recv_sems.at[recv_idx],
                device_id=(p ^ partner_m,),
                device_id_type=pl.DeviceIdType.MESH,
            )
            d.start()
            return d

        def recv_wait(dst_off, n, recv_idx):
            d = pltpu.make_async_remote_copy(
                src_ref=abuf_ref.at[pl.ds(dst_off, n)],
                dst_ref=abuf_ref.at[pl.ds(dst_off, n)],
                send_sem=rs_sem,
                recv_sem=recv_sems.at[recv_idx],
                device_id=(p,),
                device_id_type=pl.DeviceIdType.MESH,
            )
            d.wait_recv()

        sends = [
            ag_send(r0, 128, 2, 0, 4),
            ag_send(r0, 128, 4, 1, 5),
            ag_send(r0, 128, 1, 2, 6),
        ]
        out_band(r0, 128)

        recv_wait(p0, 128, 4)
        sends.append(ag_send(p0, 128, 4, 3, 5))
        sends.append(ag_send(p0, 128, 1, 4, 6))
        out_band(p0, 128)

        recv_wait(p1, 256, 5)
        sends.append(ag_send(p1, 256, 1, 5, 6))
        out_band(p1, 256)

        recv_wait(p2, 512, 6)
        out_band(p2, 512)

        for d in sends:
            d.wait_send()

    grid_spec = None
    return pl.pallas_call(
        body,
        out_shape=jax.ShapeDtypeStruct((SQ, DM), F32),
        in_specs=[pl.BlockSpec(memory_space=pltpu.VMEM)] * 5,
        out_specs=pl.BlockSpec(memory_space=pltpu.VMEM),
        scratch_shapes=[
            pltpu.VMEM((SQ, DM), BF16),
            pltpu.VMEM((SQ, W), F32),
            pltpu.VMEM((SQ, W), BF16),
            pltpu.VMEM((512, W), BF16),
            pltpu.VMEM((256, W), BF16),
            pltpu.VMEM((128, W), BF16),
            pltpu.VMEM((128, W), BF16),
            pltpu.SemaphoreType.DMA,
            pltpu.SemaphoreType.DMA((6,)),
            pltpu.SemaphoreType.DMA((7,)),
        ],
        compiler_params=pltpu.CompilerParams(collective_id=0),
    )(x2d, wq, k, v, wo)


def kernel(x, Wq, K_ext, V_ext, Wo):
    x2d = x[0].astype(BF16)
    wq = Wq.astype(BF16)
    k = K_ext[0].reshape(SKV_LOC, DM).astype(BF16)
    v = V_ext[0].reshape(SKV_LOC, DM).astype(BF16)
    wo = Wo.astype(BF16)
    out = _fused(x2d, wq, k, v, wo)
    return out.reshape(1, SQ, DM)


# device time: 81914 ns/iter; 3.4509x vs baseline; 1.0184x over previous
import jax
import jax.numpy as jnp
from jax import lax
from jax.experimental import pallas as pl
from jax.experimental.pallas import tpu as pltpu

N_DEV = 16
SQ = 1024
SKV_LOC = 1024
HQ = 8
DH = 128
DM = HQ * DH
BLK = 64
W = DM + 128
SCALE = 0.08838834764831843
KV_BLOCKS_LOC = SKV_LOC // BLK

MASKS_RS = [1, 4, 2]
SHIFTS_RS = [0, 2, 1]
MASKS_AG = [2, 4, 1]
BF16 = jnp.bfloat16
F32 = jnp.float32


def _fused(x2d, wq, k, v, wo):

    def body(x_ref, wq_ref, k_ref, v_ref, wo_ref, out_ref,
             qbuf, acc_ref, abuf_ref, land0, land1, land2, land3,
             rs_sem, ag_sems, recv_sems):
        p = lax.axis_index("i")

        barrier = pltpu.get_barrier_semaphore()
        for m in (1, 2, 4, 8):
            pl.semaphore_signal(barrier, inc=1, device_id=(p ^ m,),
                                device_id_type=pl.DeviceIdType.MESH)

        qbuf[...] = (jnp.dot(x_ref[...], wq_ref[...],
                             preferred_element_type=F32) * SCALE).astype(BF16)

        def compute_band(off, n):
            rows = lax.broadcasted_iota(jnp.int32, (n, SKV_LOC), 0) + off
            cols = lax.broadcasted_iota(jnp.int32, (n, SKV_LOC), 1)
            qb = rows // BLK
            jg = p * KV_BLOCKS_LOC + cols // BLK
            mask = (qb == jg) | (jg == 0) | (lax.rem(qb + jg, 3) == 0)
            ls = []
            for h in range(HQ):
                qh = qbuf[pl.ds(off, n), pl.ds(h * DH, DH)]
                kh = k_ref[:, h * DH:(h + 1) * DH]
                vh = v_ref[:, h * DH:(h + 1) * DH]
                s = lax.dot_general(qh, kh, (((1,), (1,)), ((), ())),
                                    preferred_element_type=F32)
                w = jnp.where(mask, jnp.exp(s), 0.0)
                ls.append(jnp.sum(w, axis=1, keepdims=True))
                acc_ref[pl.ds(off, n), pl.ds(h * DH, DH)] = jnp.dot(
                    w.astype(BF16), vh, preferred_element_type=F32)
            acc_ref[pl.ds(off, n), pl.ds(DM, 128)] = jnp.concatenate(
                ls + [jnp.zeros((n, 128 - HQ), F32)], axis=1)

        def out_band(off, n):
            lb = abuf_ref[pl.ds(off, n), pl.ds(DM, 128)].astype(F32)
            o = jnp.zeros((n, DM), F32)
            for h in range(HQ):
                ch = abuf_ref[pl.ds(off, n), pl.ds(h * DH, DH)].astype(F32)
                nh = (ch / lb[:, h:h + 1]).astype(BF16)
                o = o + jnp.dot(nh, wo_ref[h * DH:(h + 1) * DH, :],
                                preferred_element_type=F32)
            out_ref[pl.ds(off, n), :] = o

        b0 = p & 1
        send_start = pl.multiple_of((1 - b0) * 512, 64)
        keep_start = pl.multiple_of(b0 * 512, 64)
        compute_band(send_start, 512)
        abuf_ref[pl.ds(send_start, 512)] = (
            acc_ref[pl.ds(send_start, 512)].astype(BF16))
        pl.semaphore_wait(barrier, 4)
        rs0 = pltpu.make_async_remote_copy(
            src_ref=abuf_ref.at[pl.ds(send_start, 512)],
            dst_ref=land0,
            send_sem=rs_sem,
            recv_sem=recv_sems.at[0],
            device_id=(p ^ 1,),
            device_id_type=pl.DeviceIdType.MESH,
        )
        rs0.start()
        compute_band(keep_start, 512)
        rs0.wait_recv()
        acc_ref[pl.ds(keep_start, 512)] = (
            acc_ref[pl.ds(keep_start, 512)] + land0[...].astype(F32))
        prev_send = rs0
        start = keep_start
        size = 512

        lands = [land0, land1, land2]
        for j, (m, sh) in enumerate(zip(MASKS_RS[1:], SHIFTS_RS[1:]), 1):
            half = size // 2
            b = (p >> sh) & 1
            keep_start = pl.multiple_of(start + b * half, 64)
            send_start = pl.multiple_of(start + (1 - b) * half, 64)
            abuf_ref[pl.ds(send_start, half)] = (
                acc_ref[pl.ds(send_start, half)].astype(BF16))
            rdma = pltpu.make_async_remote_copy(
                src_ref=abuf_ref.at[pl.ds(send_start, half)],
                dst_ref=lands[j],
                send_sem=rs_sem,
                recv_sem=recv_sems.at[j],
                device_id=(p ^ m,),
                device_id_type=pl.DeviceIdType.MESH,
            )
            prev_send.wait_send()
            rdma.start()
            prev_send = rdma
            rdma.wait_recv()
            acc_ref[pl.ds(keep_start, half)] = (
                acc_ref[pl.ds(keep_start, half)]
                + lands[j][...].astype(F32))
            start = keep_start
            size = half

        start = pl.multiple_of(start, 64)
        abuf_ref[pl.ds(start, 128)] = acc_ref[pl.ds(start, 128)].astype(BF16)
        prev_send.wait_send()
        xadd = pltpu.make_async_remote_copy(
            src_ref=abuf_ref.at[pl.ds(start, 128)],
            dst_ref=land3,
            send_sem=rs_sem,
            recv_sem=recv_sems.at[3],
            device_id=(p ^ 8,),
            device_id_type=pl.DeviceIdType.MESH,
        )
        xadd.start()
        xadd.wait_recv()
        xadd.wait_send()
        abuf_ref[pl.ds(start, 128)] = (
            acc_ref[pl.ds(start, 128)] + land3[...].astype(F32)).astype(BF16)

        def reg(pp):
            return pl.multiple_of(
                (pp & 1) * 512 + ((pp >> 2) & 1) * 256 + ((pp >> 1) & 1) * 128,
                64)

        a_off = reg(p ^ 2)
        b1_off = reg(p ^ 4)
        b2_off = reg(p ^ 4 ^ 2)
        c1_off = reg(p ^ 1)
        c2_off = reg(p ^ 1 ^ 2)
        q2 = p ^ 1
        c3_off = pl.multiple_of(
            (q2 & 1) * 512 + (1 - ((q2 >> 2) & 1)) * 256, 64)

        def ag_send(src_off, n, partner_m, sem_idx, recv_idx):
            d = pltpu.make_async_remote_copy(
                src_ref=abuf_ref.at[pl.ds(src_off, n)],
                dst_ref=abuf_ref.at[pl.ds(src_off, n)],
                send_sem=ag_sems.at[sem_idx],
                recv_sem=recv_sems.at[recv_idx],
                device_id=(p ^ partner_m,),
                device_id_type=pl.DeviceIdType.MESH,
            )
            d.start()
            return d

        def recv_wait(dst_off, n, recv_idx):
            d = pltpu.make_async_remote_copy(
                src_ref=abuf_ref.at[pl.ds(dst_off, n)],
                dst_ref=abuf_ref.at[pl.ds(dst_off, n)],
                send_sem=rs_sem,
                recv_sem=recv_sems.at[recv_idx],
                device_id=(p,),
                device_id_type=pl.DeviceIdType.MESH,
            )
            d.wait_recv()

        r0 = start
        sends = [
            ag_send(r0, 128, 2, 0, 4),
            ag_send(r0, 128, 4, 1, 5),
            ag_send(r0, 128, 1, 2, 7),
        ]
        out_band(r0, 128)

        recv_wait(a_off, 128, 4)
        sends.append(ag_send(a_off, 128, 4, 3, 6))
        sends.append(ag_send(a_off, 128, 1, 4, 8))
        out_band(a_off, 128)

        recv_wait(b1_off, 128, 5)
        sends.append(ag_send(b1_off, 128, 1, 5, 9))
        out_band(b1_off, 128)

        recv_wait(b2_off, 128, 6)
        sends.append(ag_send(b2_off, 128, 1, 6, 9))
        out_band(b2_off, 128)

        recv_wait(c1_off, 128, 7)
        out_band(c1_off, 128)
        recv_wait(c2_off, 128, 8)
        out_band(c2_off, 128)
        recv_wait(c3_off, 256, 9)
        out_band(c3_off, 256)

        for d in sends:
            d.wait_send()

    grid_spec = None
    return pl.pallas_call(
        body,
        out_shape=jax.ShapeDtypeStruct((SQ, DM), F32),
        in_specs=[pl.BlockSpec(memory_space=pltpu.VMEM)] * 5,
        out_specs=pl.BlockSpec(memory_space=pltpu.VMEM),
        scratch_shapes=[
            pltpu.VMEM((SQ, DM), BF16),
            pltpu.VMEM((SQ, W), F32),
            pltpu.VMEM((SQ, W), BF16),
            pltpu.VMEM((512, W), BF16),
            pltpu.VMEM((256, W), BF16),
            pltpu.VMEM((128, W), BF16),
            pltpu.VMEM((128, W), BF16),
            pltpu.SemaphoreType.DMA,
            pltpu.SemaphoreType.DMA((7,)),
            pltpu.SemaphoreType.DMA((10,)),
        ],
        compiler_params=pltpu.CompilerParams(collective_id=0),
    )(x2d, wq, k, v, wo)


def kernel(x, Wq, K_ext, V_ext, Wo):
    x2d = x[0].astype(BF16)
    wq = Wq.astype(BF16)
    k = K_ext[0].reshape(SKV_LOC, DM).astype(BF16)
    v = V_ext[0].reshape(SKV_LOC, DM).astype(BF16)
    wo = Wo.astype(BF16)
    out = _fused(x2d, wq, k, v, wo)
    return out.reshape(1, SQ, DM)
